# Initial kernel scaffold; baseline (speedup 1.0000x reference)
import jax
import jax.numpy as jnp
from jax.experimental import pallas as pl


def kernel(x, y, W1, b1, W2, b2, W3, b3):
    raise NotImplementedError("write your pallas kernel here")


# speedup vs baseline: 85.1797x; 85.1797x over previous
import functools

import jax
import jax.numpy as jnp
import numpy as np
from jax import lax
from jax.experimental import pallas as pl
from jax.experimental.pallas import tpu as pltpu
from jax.experimental.pallas import tpu_sc as plsc

LEAKY = 0.1
K = 5
WS = 20
Q = 15876
NB = 63
BQ = 252
GW = 1240
NY = 3844
F = 72

NQK = Q * K
NCH = 4
CHUNK = 19872
NQK_PAD = NCH * CHUNK
NYP = 3848
NXP = 15880
ITEMS_PER_W = 9


RB = 2048
HWX = 128 * 128
NBX = HWX // RB


def _conv_block_body(Cin3, Cout, x_ref, w_ref, b_ref, o_ref):
    i = pl.program_id(0)
    X = x_ref[...]
    Wd = jnp.where(i < NBX, 128, 64)
    col = lax.broadcasted_iota(jnp.int32, (RB, 1), 0) % Wd
    acc = jnp.broadcast_to(b_ref[0:1, :], (RB, Cout)) * 1.0
    zrow = jnp.zeros((1, Cout), jnp.float32)
    for v in range(3):
        Wv = w_ref[v * Cin3:(v + 1) * Cin3, :]
        T = jnp.dot(X, Wv, preferred_element_type=jnp.float32,
                    precision=lax.Precision.HIGHEST)
        if v == 0:
            Ts = jnp.concatenate([zrow, T[:-1]], axis=0)
            Ts = jnp.where(col == 0, 0.0, Ts)
        elif v == 2:
            Ts = jnp.concatenate([T[1:], zrow], axis=0)
            Ts = jnp.where(col == Wd - 1, 0.0, Ts)
        else:
            Ts = T
        acc = acc + Ts
    acc = jnp.where(acc >= 0, acc, LEAKY * acc)
    o_ref[...] = acc


def _conv_layer(Xc, wm, bm, interpret=False):
    HW2, Cin3x3 = Xc.shape
    Cout = wm.shape[1]
    return pl.pallas_call(
        functools.partial(_conv_block_body, Cin3x3, Cout),
        grid=(HW2 // RB,),
        in_specs=[
            pl.BlockSpec((RB, Cin3x3), lambda i: (i, 0)),
            pl.BlockSpec(wm.shape, lambda i: (0, 0)),
            pl.BlockSpec((1, Cout), lambda i: (0, 0)),
        ],
        out_specs=pl.BlockSpec((RB, Cout), lambda i: (i, 0)),
        out_shape=jax.ShapeDtypeStruct((HW2, Cout), jnp.float32),
        compiler_params=pltpu.CompilerParams(vmem_limit_bytes=50 * 1024 * 1024),
        interpret=interpret,
    )(Xc, wm, bm)


def _cat3(E, Wd):
    C = E.shape[1]
    z = jnp.zeros((Wd, C), jnp.float32)
    dn = jnp.concatenate([z, E[:-Wd]], axis=0)
    up = jnp.concatenate([E[Wd:], z], axis=0)
    return jnp.concatenate([dn, E, up], axis=1)


def _embed_both(Xf, Yf, wb, interpret=False):
    h = jnp.concatenate([_cat3(Xf, 128), _cat3(Yf, 64)], axis=0)
    for li in range(3):
        wm, bm = wb[2 * li], wb[2 * li + 1]
        h = _conv_layer(h, wm, bm, interpret=interpret)
        if li < 2:
            h = jnp.concatenate([_cat3(h[:HWX], 128), _cat3(h[HWX:], 64)],
                                axis=0)
    return h


def _prep_weights(W1, b1, W2, b2, W3, b3):
    out = []
    for W_, b_ in ((W1, b1), (W2, b2), (W3, b3)):
        Cin = W_.shape[1]
        out.append(jnp.transpose(W_, (3, 2, 1, 0)).reshape(9 * Cin, W_.shape[0]))
        out.append(b_[None, :])
    return tuple(out)


def _patches_qmajor(E, H, W):
    Hp, Wp = H - 2, W - 2
    E3 = E.reshape(H, W, 8)
    sl = [E3[u:u + Hp, v:v + Wp, :].reshape(Hp * Wp, 8)
          for u in range(3) for v in range(3)]
    t = jnp.stack(sl, axis=2)
    return t.reshape(Hp * Wp, F)


def _dist_topk_body(x_ref, g_ref, s_ref, i_ref):
    b = pl.program_id(0)
    r0 = jnp.clip(b - 10, 0, 42)
    x = x_ref[0]
    g = g_ref[0]
    S = lax.dot_general(x, g, (((1,), (1,)), ((), ())),
                        preferred_element_type=jnp.float32,
                        precision=lax.Precision.HIGHEST)
    x2 = jnp.sum(x * x, axis=1, keepdims=True)
    ones_r = jnp.ones((1, F), jnp.float32)
    g2 = lax.dot_general(ones_r, g * g, (((1,), (1,)), ((), ())),
                         preferred_element_type=jnp.float32,
                         precision=lax.Precision.HIGHEST)
    dist = x2 - 2.0 * S + g2

    col = lax.broadcasted_iota(jnp.int32, (BQ, GW), 1)
    c_of = col % 62
    row = lax.broadcasted_iota(jnp.int32, (BQ, 1), 0)
    qj = row % 126
    c0 = jnp.clip(qj // 2 - 10, 0, 42)
    dc = c_of - c0
    inf = jnp.float32(np.inf)
    dist = jnp.where((dc >= 0) & (dc < WS), dist, inf)

    base = r0 * 62
    for k in range(K):
        m = jnp.min(dist, axis=1, keepdims=True)
        lidx = jnp.min(jnp.where(dist == m, col, 1 << 30),
                       axis=1, keepdims=True)
        s_ref[0, :, k:k + 1] = -m
        i_ref[0, :, k:k + 1] = lidx + base
        dist = jnp.where(col == lidx, inf, dist)


def _dist_topk(xp3, ypwin, interpret=False):
    return pl.pallas_call(
        _dist_topk_body,
        grid=(NB,),
        in_specs=[
            pl.BlockSpec((1, BQ, F), lambda b: (b, 0, 0)),
            pl.BlockSpec((1, GW, F), lambda b: (b, 0, 0)),
        ],
        out_specs=[
            pl.BlockSpec((1, BQ, K), lambda b: (b, 0, 0)),
            pl.BlockSpec((1, BQ, K), lambda b: (b, 0, 0)),
        ],
        out_shape=[
            jax.ShapeDtypeStruct((NB, BQ, K), jnp.float32),
            jax.ShapeDtypeStruct((NB, BQ, K), jnp.int32),
        ],
        interpret=interpret,
    )(xp3, ypwin)


def _sc_diff(ypf, xpf, idxc, qidxc):
    mesh = plsc.VectorSubcoreMesh(core_axis_name="c", subcore_axis_name="s")

    @functools.partial(
        pl.kernel,
        mesh=mesh,
        compiler_params=pltpu.CompilerParams(needs_layout_passes=False),
        out_type=jax.ShapeDtypeStruct((F, NCH, CHUNK), jnp.float32),
        scratch_types=[
            pltpu.VMEM((NYP,), jnp.float32),
            pltpu.VMEM((NXP,), jnp.float32),
            pltpu.VMEM((CHUNK,), jnp.int32),
            pltpu.VMEM((CHUNK,), jnp.int32),
            pltpu.VMEM((CHUNK,), jnp.float32),
        ],
    )
    def k(ypf_hbm, xpf_hbm, idx_hbm, qidx_hbm, out_hbm,
          yrow_v, xrow_v, idx_v, qidx_v, out_v):
        wid = lax.axis_index("s") * 2 + lax.axis_index("c")

        def item_body(t, carry):
            item = wid * ITEMS_PER_W + t
            f = item // NCH
            j = item % NCH
            pltpu.sync_copy(ypf_hbm.at[f], yrow_v)
            pltpu.sync_copy(xpf_hbm.at[f], xrow_v)
            pltpu.sync_copy(idx_hbm.at[j], idx_v)
            pltpu.sync_copy(qidx_hbm.at[j], qidx_v)

            def vec_body(i, c):
                iv = idx_v[pl.ds(i * 16, 16)]
                yv = plsc.load_gather(yrow_v, [iv])
                qv = qidx_v[pl.ds(i * 16, 16)]
                xv = plsc.load_gather(xrow_v, [qv])
                out_v[pl.ds(i * 16, 16)] = xv - yv
                return c

            lax.fori_loop(0, CHUNK // 16, vec_body, 0)
            pltpu.sync_copy(out_v, out_hbm.at[f, j])
            return carry

        lax.fori_loop(0, ITEMS_PER_W, item_body, 0)

    return k(ypf, xpf, idxc, qidxc)


def _lrelu(v):
    return jnp.where(v >= 0, v, LEAKY * v)


def _conv2d(v, w, b):
    o = lax.conv_general_dilated(v, w, (1, 1), 'SAME',
                                 dimension_numbers=('NCHW', 'OIHW', 'NCHW'))
    return o + b[None, :, None, None]


def kernel(x, y, W1, b1, W2, b2, W3, b3):
    def emb(img):
        h = _lrelu(_conv2d(img, W1, b1))
        h = _lrelu(_conv2d(h, W2, b2))
        h = _lrelu(_conv2d(h, W3, b3))
        return h

    xe4 = emb(x)
    ye4 = emb(y)
    xpT = lax.conv_general_dilated_patches(
        xe4, (3, 3), (1, 1), 'VALID').reshape(F, Q).T
    ypT = lax.conv_general_dilated_patches(
        ye4, (3, 3), (1, 1), 'VALID').reshape(F, NY).T

    r0s = jnp.clip(jnp.arange(NB) - 10, 0, 42)
    rows = r0s[:, None] * 62 + jnp.arange(GW)[None, :]
    ypwin = ypT[rows]

    score, idx = _dist_topk(xpT.reshape(NB, BQ, F), ypwin)
    score = score.reshape(Q, K)
    idx = idx.reshape(Q, K)

    ypf = jnp.zeros((F, NYP), jnp.float32).at[:, :NY].set(ypT.T)
    xpf = jnp.zeros((F, NXP), jnp.float32).at[:, :Q].set(xpT.T)
    idxc = jnp.zeros((NQK_PAD,), jnp.int32).at[:NQK].set(
        idx.reshape(-1)).reshape(NCH, CHUNK)
    qidxc = jnp.minimum(jnp.arange(NQK_PAD, dtype=jnp.int32) // K,
                        Q - 1).reshape(NCH, CHUNK)
    diff = _sc_diff(ypf, xpf, idxc, qidxc)
    diff = diff.reshape(F, NQK_PAD)[:, :NQK].reshape(1, F, Q, K)

    return score[None], idx[None], diff


# speedup vs baseline: 85.3619x; 1.0021x over previous
import functools

import jax
import jax.numpy as jnp
import numpy as np
from jax import lax
from jax.experimental import pallas as pl
from jax.experimental.pallas import tpu as pltpu
from jax.experimental.pallas import tpu_sc as plsc

LEAKY = 0.1
K = 5
WS = 20
Q = 15876
NB = 63
BQ = 252
GW = 1240
NY = 3844
F = 72

NQK = Q * K
NCH = 4
CHUNK = 19872
NQK_PAD = NCH * CHUNK
NYP = 3848
NXP = 15880
ITEMS_PER_W = 9


def _dist_topk_body(x_ref, g_ref, s_ref, i_ref):
    b = pl.program_id(0)
    r0 = jnp.clip(b - 10, 0, 42)
    x = x_ref[0]
    g = g_ref[0]
    S = lax.dot_general(x, g, (((1,), (1,)), ((), ())),
                        preferred_element_type=jnp.float32,
                        precision=lax.Precision.HIGHEST)
    x2 = jnp.sum(x * x, axis=1, keepdims=True)
    ones_r = jnp.ones((1, F), jnp.float32)
    g2 = lax.dot_general(ones_r, g * g, (((1,), (1,)), ((), ())),
                         preferred_element_type=jnp.float32,
                         precision=lax.Precision.HIGHEST)
    dist = x2 - 2.0 * S + g2

    col = lax.broadcasted_iota(jnp.int32, (BQ, GW), 1)
    c_of = col % 62
    row = lax.broadcasted_iota(jnp.int32, (BQ, 1), 0)
    qj = row % 126
    c0 = jnp.clip(qj // 2 - 10, 0, 42)
    dc = c_of - c0
    inf = jnp.float32(np.inf)
    dist = jnp.where((dc >= 0) & (dc < WS), dist, inf)

    base = r0 * 62
    for k in range(K):
        m = jnp.min(dist, axis=1, keepdims=True)
        lidx = jnp.min(jnp.where(dist == m, col, 1 << 30),
                       axis=1, keepdims=True)
        s_ref[0, :, k:k + 1] = -m
        i_ref[0, :, k:k + 1] = lidx + base
        dist = jnp.where(col == lidx, inf, dist)


def _dist_topk(xp3, ypwin, interpret=False):
    return pl.pallas_call(
        _dist_topk_body,
        grid=(NB,),
        in_specs=[
            pl.BlockSpec((1, BQ, F), lambda b: (b, 0, 0)),
            pl.BlockSpec((1, GW, F), lambda b: (b, 0, 0)),
        ],
        out_specs=[
            pl.BlockSpec((1, BQ, K), lambda b: (b, 0, 0)),
            pl.BlockSpec((1, BQ, K), lambda b: (b, 0, 0)),
        ],
        out_shape=[
            jax.ShapeDtypeStruct((NB, BQ, K), jnp.float32),
            jax.ShapeDtypeStruct((NB, BQ, K), jnp.int32),
        ],
        interpret=interpret,
    )(xp3, ypwin)


def _sc_diff(ypf, xpf, idxc, qidxc):
    mesh = plsc.VectorSubcoreMesh(core_axis_name="c", subcore_axis_name="s")

    @functools.partial(
        pl.kernel,
        mesh=mesh,
        compiler_params=pltpu.CompilerParams(needs_layout_passes=False),
        out_type=jax.ShapeDtypeStruct((F, NCH, CHUNK), jnp.float32),
        scratch_types=[
            pltpu.VMEM((NYP,), jnp.float32),
            pltpu.VMEM((NXP,), jnp.float32),
            pltpu.VMEM((CHUNK,), jnp.int32),
            pltpu.VMEM((CHUNK,), jnp.int32),
            pltpu.VMEM((CHUNK,), jnp.float32),
        ],
    )
    def k(ypf_hbm, xpf_hbm, idx_hbm, qidx_hbm, out_hbm,
          yrow_v, xrow_v, idx_v, qidx_v, out_v):
        wid = lax.axis_index("s") * 2 + lax.axis_index("c")

        def item_body(t, carry):
            item = wid * ITEMS_PER_W + t
            f = item // NCH
            j = item % NCH
            pltpu.sync_copy(ypf_hbm.at[f], yrow_v)
            pltpu.sync_copy(xpf_hbm.at[f], xrow_v)
            pltpu.sync_copy(idx_hbm.at[j], idx_v)
            pltpu.sync_copy(qidx_hbm.at[j], qidx_v)

            def vec_body(i, c):
                iv = idx_v[pl.ds(i * 16, 16)]
                yv = plsc.load_gather(yrow_v, [iv])
                qv = qidx_v[pl.ds(i * 16, 16)]
                xv = plsc.load_gather(xrow_v, [qv])
                out_v[pl.ds(i * 16, 16)] = xv - yv
                return c

            lax.fori_loop(0, CHUNK // 16, vec_body, 0)
            pltpu.sync_copy(out_v, out_hbm.at[f, j])
            return carry

        lax.fori_loop(0, ITEMS_PER_W, item_body, 0)

    return k(ypf, xpf, idxc, qidxc)


def _lrelu(v):
    return jnp.where(v >= 0, v, LEAKY * v)


def _conv2d(v, w, b):
    o = lax.conv_general_dilated(v, w, (1, 1), 'SAME',
                                 dimension_numbers=('NCHW', 'OIHW', 'NCHW'))
    return o + b[None, :, None, None]


def kernel(x, y, W1, b1, W2, b2, W3, b3):
    def emb(img):
        h = _lrelu(_conv2d(img, W1, b1))
        h = _lrelu(_conv2d(h, W2, b2))
        h = _lrelu(_conv2d(h, W3, b3))
        return h

    xe4 = emb(x)
    ye4 = emb(y)
    xpT = lax.conv_general_dilated_patches(
        xe4, (3, 3), (1, 1), 'VALID').reshape(F, Q).T
    ypT = lax.conv_general_dilated_patches(
        ye4, (3, 3), (1, 1), 'VALID').reshape(F, NY).T

    r0s = jnp.clip(jnp.arange(NB) - 10, 0, 42)
    rows = r0s[:, None] * 62 + jnp.arange(GW)[None, :]
    ypwin = ypT[rows]

    score, idx = _dist_topk(xpT.reshape(NB, BQ, F), ypwin)
    score = score.reshape(Q, K)
    idx = idx.reshape(Q, K)

    ypf = jnp.zeros((F, NYP), jnp.float32).at[:, :NY].set(ypT.T)
    xpf = jnp.zeros((F, NXP), jnp.float32).at[:, :Q].set(xpT.T)
    idxc = jnp.zeros((NQK_PAD,), jnp.int32).at[:NQK].set(
        idx.reshape(-1)).reshape(NCH, CHUNK)
    qidxc = jnp.minimum(jnp.arange(NQK_PAD, dtype=jnp.int32) // K,
                        Q - 1).reshape(NCH, CHUNK)
    diff = _sc_diff(ypf, xpf, idxc, qidxc)
    diff = diff.reshape(F, NQK_PAD)[:, :NQK].reshape(1, F, Q, K)

    return score[None], idx[None], diff
